# baseline (speedup 1.0000x reference)
import functools
import math

import jax
import jax.numpy as jnp
from jax import lax
from jax.experimental import pallas as pl
from jax.experimental.pallas import tpu as pltpu
from jax.experimental.pallas import tpu_sc as plsc

D_MODEL = 64
SCALE = math.sqrt(D_MODEL)
LANES = 16
BB = 128


@functools.lru_cache(maxsize=None)
def _build(S, NB):
    info = plsc.get_sparse_core_info()
    NC, NS = info.num_cores, info.num_subcores
    NW = NC * NS
    n_units = S * NB
    assert n_units % NW == 0
    U = n_units // NW
    assert U % 2 == 0 and U >= 6

    mesh = plsc.VectorSubcoreMesh(core_axis_name="c", subcore_axis_name="s")

    @functools.partial(
        pl.kernel,
        mesh=mesh,
        out_type=jax.ShapeDtypeStruct(
            (S, D_MODEL // 8, NB, 8, BB), jnp.float32
        ),
        compiler_params=pltpu.CompilerParams(
            use_tc_tiling_on_sc=False, needs_layout_passes=False
        ),
        scratch_types=[
            pltpu.VMEM((BB,), jnp.int32),
            pltpu.VMEM((BB,), jnp.int32),
            pltpu.VMEM((BB, D_MODEL), jnp.float32),
            pltpu.VMEM((BB, D_MODEL), jnp.float32),
            pltpu.VMEM((BB, D_MODEL + 1), jnp.float32),
            pltpu.VMEM((BB, D_MODEL + 1), jnp.float32),
            pltpu.VMEM((D_MODEL // 8, 8, BB), jnp.float32),
            pltpu.VMEM((D_MODEL // 8, 8, BB), jnp.float32),
            pltpu.SemaphoreType.DMA,
            pltpu.SemaphoreType.DMA,
            pltpu.SemaphoreType.DMA,
            pltpu.SemaphoreType.DMA,
            pltpu.SemaphoreType.DMA,
            pltpu.SemaphoreType.DMA,
        ],
    )
    def emb(xtl_hbm, table_hbm, out_hbm, ibuf0, ibuf1, g0, g1, gp0, gp1,
            o0, o1, isem0, isem1, gsem0, gsem1, osem0, osem1):
        ibuf = (ibuf0, ibuf1)
        gb = (g0, g1)
        gp = (gp0, gp1)
        ob = (o0, o1)
        isem = (isem0, isem1)
        gsem = (gsem0, gsem1)
        osem = (osem0, osem1)

        wid = lax.axis_index("s") * NC + lax.axis_index("c")
        u0 = wid * U

        rowv = [
            lax.broadcasted_iota(jnp.int32, (LANES,), 0) + g * LANES
            for g in range(BB // LANES)
        ]

        def unit_su(u):
            ug = u0 + u
            return ug // NB, lax.rem(ug, NB)

        def issue_idx(u, b):
            s, bhi = unit_su(u)
            pltpu.async_copy(
                xtl_hbm.at[pl.ds(s * (NB * BB) + bhi * BB, BB)],
                ibuf[b], isem[b],
            )

        def wait_idx(u, b):
            s, bhi = unit_su(u)
            pltpu.make_async_copy(
                xtl_hbm.at[pl.ds(s * (NB * BB) + bhi * BB, BB)],
                ibuf[b], isem[b],
            ).wait()

        def issue_gather(b):
            pltpu.async_copy(table_hbm.at[ibuf[b]], gb[b], gsem[b])

        def wait_gather(b):
            pltpu.make_async_copy(
                table_hbm.at[ibuf[b]], gb[b], gsem[b]
            ).wait()

        def issue_out(u, b):
            s, bhi = unit_su(u)
            pltpu.async_copy(ob[b], out_hbm.at[s, :, bhi], osem[b])

        def wait_out(u, b):
            s, bhi = unit_su(u)
            pltpu.make_async_copy(
                ob[b], out_hbm.at[s, :, bhi], osem[b]
            ).wait()

        def transpose(b):
            @plsc.parallel_loop(0, BB, unroll=8)
            def rp(r):
                for j in range(D_MODEL // LANES):
                    sl = pl.ds(j * LANES, LANES)
                    gp[b][r, sl] = gb[b][r, sl]

            @plsc.parallel_loop(0, D_MODEL, unroll=8)
            def tp(d):
                dhi = lax.shift_right_logical(d, 3)
                dlo = lax.bitwise_and(d, 7)
                cv = jnp.full((LANES,), d, jnp.int32)
                for g in range(BB // LANES):
                    v = plsc.load_gather(gp[b], [rowv[g], cv])
                    ob[b][dhi, dlo, pl.ds(g * LANES, LANES)] = v * SCALE

        issue_idx(0, 0)
        issue_idx(1, 1)
        wait_idx(0, 0)
        issue_gather(0)
        for u in range(2):
            b = u % 2
            wait_gather(b)
            wait_idx(u + 1, 1 - b)
            issue_gather(1 - b)
            issue_idx(u + 2, b)
            transpose(b)
            issue_out(u, b)

        def pair(i, carry):
            for b in range(2):
                u = 2 * i + b
                wait_gather(b)
                wait_idx(u + 1, 1 - b)
                issue_gather(1 - b)
                issue_idx(u + 2, b)
                wait_out(u - 2, b)
                transpose(b)
                issue_out(u, b)
            return carry

        lax.fori_loop(1, U // 2 - 1, pair, 0)

        u = U - 2
        wait_gather(0)
        wait_idx(u + 1, 1)
        issue_gather(1)
        wait_out(u - 2, 0)
        transpose(0)
        issue_out(u, 0)

        u = U - 1
        wait_gather(1)
        wait_out(u - 2, 1)
        transpose(1)
        issue_out(u, 1)

        wait_out(U - 2, 0)
        wait_out(U - 1, 1)

    return emb


@functools.lru_cache(maxsize=None)
def _build_fmt(V):
    info = plsc.get_sparse_core_info()
    NC, NS = info.num_cores, info.num_subcores
    NW = NC * NS
    W = 256
    NF = V // W
    NU = NF + 1
    UPW = (NU + NW - 1) // NW
    assert V - NF * W == D_MODEL

    mesh = plsc.VectorSubcoreMesh(core_axis_name="c", subcore_axis_name="s")

    @functools.partial(
        pl.kernel,
        mesh=mesh,
        out_type=jax.ShapeDtypeStruct((V // 2, 2 * D_MODEL), jnp.float32),
        compiler_params=pltpu.CompilerParams(
            use_tc_tiling_on_sc=True, needs_layout_passes=False
        ),
        name="fmt",
        scratch_types=[
            pltpu.VMEM((D_MODEL, W), jnp.float32),
            pltpu.VMEM((D_MODEL, W), jnp.float32),
            pltpu.VMEM((D_MODEL, W + 1), jnp.float32),
            pltpu.VMEM((W // 2, 2 * D_MODEL), jnp.float32),
            pltpu.VMEM((W // 2, 2 * D_MODEL), jnp.float32),
            pltpu.SemaphoreType.DMA,
            pltpu.SemaphoreType.DMA,
            pltpu.SemaphoreType.DMA,
            pltpu.SemaphoreType.DMA,
        ],
    )
    def fmt(tbt_hbm, tail_hbm, scr_hbm, s0, s1, sp, o0, o1, is0, is1,
            os0, os1):
        sb = (s0, s1)
        ob = (o0, o1)
        isem = (is0, is1)
        osem = (os0, os1)

        wid = lax.axis_index("s") * NC + lax.axis_index("c")

        rowv = [
            lax.broadcasted_iota(jnp.int32, (LANES,), 0) + g * LANES
            for g in range(D_MODEL // LANES)
        ]

        def unit(i):
            return wid + NW * i

        def dma_in(u, b, issue):
            @pl.when(u < NF)
            def _():
                c = pltpu.async_copy if issue else (
                    lambda s, d, m: pltpu.make_async_copy(s, d, m).wait())
                c(tbt_hbm.at[:, pl.ds(u * W, W)], sb[b], isem[b])

            @pl.when(u == NF)
            def _():
                c = pltpu.async_copy if issue else (
                    lambda s, d, m: pltpu.make_async_copy(s, d, m).wait())
                c(tail_hbm, sb[b].at[:, pl.ds(0, BB)], isem[b])

        def dma_out(u, b, issue):
            @pl.when(u < NF)
            def _():
                c = pltpu.async_copy if issue else (
                    lambda s, d, m: pltpu.make_async_copy(s, d, m).wait())
                c(ob[b], scr_hbm.at[pl.ds(u * (W // 2), W // 2)], osem[b])

            @pl.when(u == NF)
            def _():
                c = pltpu.async_copy if issue else (
                    lambda s, d, m: pltpu.make_async_copy(s, d, m).wait())
                c(
                    ob[b].at[pl.ds(0, D_MODEL // 2)],
                    scr_hbm.at[pl.ds(NF * (W // 2), D_MODEL // 2)],
                    osem[b],
                )

        def compute(b):
            @plsc.parallel_loop(0, D_MODEL, unroll=8)
            def rp(d):
                for j in range(W // LANES):
                    sl = pl.ds(j * LANES, LANES)
                    sp[d, sl] = sb[b][d, sl]

            @plsc.parallel_loop(0, W // 2, unroll=8)
            def tp(k):
                c0 = jnp.full((LANES,), 2 * k, jnp.int32)
                c1 = jnp.full((LANES,), 2 * k + 1, jnp.int32)
                for g in range(D_MODEL // LANES):
                    v0 = plsc.load_gather(sp, [rowv[g], c0])
                    v1 = plsc.load_gather(sp, [rowv[g], c1])
                    ob[b][k, pl.ds(g * LANES, LANES)] = v0
                    ob[b][k, pl.ds(D_MODEL + g * LANES, LANES)] = v1

        dma_in(unit(0), 0, True)
        dma_in(unit(1), 1, True)

        def pair(t, carry):
            for b in range(2):
                i = 2 * t + b
                u = unit(i)

                @pl.when(u < NU)
                def _():
                    dma_in(u, b, False)
                    compute(b)

                @pl.when(jnp.logical_and(i >= 2, unit(i - 2) < NU))
                def _():
                    dma_out(unit(i - 2), b, False)

                @pl.when(u < NU)
                def _():
                    dma_out(u, b, True)

                @pl.when(unit(i + 2) < NU)
                def _():
                    dma_in(unit(i + 2), b, True)
            return carry

        lax.fori_loop(0, (UPW + 3) // 2 + 1, pair, 0)

    return fmt


def kernel(x, table):
    B0, S = x.shape
    NB = B0 // BB
    V = table.shape[0]
    tbt = jnp.transpose(table)
    tail = jnp.concatenate(
        [tbt[:, V - D_MODEL:], jnp.zeros((D_MODEL, BB - D_MODEL), jnp.float32)],
        axis=1,
    )
    scr = _build_fmt(V)(tbt, tail)
    tab_lin = scr.reshape(V, D_MODEL)
    xtl = jnp.transpose(x).reshape(-1).astype(jnp.int32)
    out5 = _build(S, NB)(xtl, tab_lin)
    out = jnp.transpose(out5, (2, 4, 0, 1, 3)).reshape(B0, S, D_MODEL)
    return out


# speedup vs baseline: 1.3302x; 1.3302x over previous
import functools
import math

import jax
import jax.numpy as jnp
from jax import lax
from jax.experimental import pallas as pl
from jax.experimental.pallas import tpu as pltpu
from jax.experimental.pallas import tpu_sc as plsc

D_MODEL = 64
SCALE = math.sqrt(D_MODEL)
LANES = 16
BB = 128


@functools.lru_cache(maxsize=None)
def _build(S, NB):
    info = plsc.get_sparse_core_info()
    NC, NS = info.num_cores, info.num_subcores
    NW = NC * NS
    n_units = S * NB
    assert n_units % NW == 0
    U = n_units // NW
    assert U % 2 == 0 and U >= 6

    mesh = plsc.VectorSubcoreMesh(core_axis_name="c", subcore_axis_name="s")

    @functools.partial(
        pl.kernel,
        mesh=mesh,
        out_type=jax.ShapeDtypeStruct(
            (S, D_MODEL // 8, NB, 8, BB), jnp.float32
        ),
        compiler_params=pltpu.CompilerParams(
            use_tc_tiling_on_sc=False, needs_layout_passes=False
        ),
        scratch_types=[
            pltpu.VMEM((BB,), jnp.int32),
            pltpu.VMEM((BB,), jnp.int32),
            pltpu.VMEM((BB, D_MODEL), jnp.float32),
            pltpu.VMEM((BB, D_MODEL), jnp.float32),
            pltpu.VMEM((BB, D_MODEL + 1), jnp.float32),
            pltpu.VMEM((BB, D_MODEL + 1), jnp.float32),
            pltpu.VMEM((D_MODEL // 8, 8, BB), jnp.float32),
            pltpu.VMEM((D_MODEL // 8, 8, BB), jnp.float32),
            pltpu.SemaphoreType.DMA,
            pltpu.SemaphoreType.DMA,
            pltpu.SemaphoreType.DMA,
            pltpu.SemaphoreType.DMA,
            pltpu.SemaphoreType.DMA,
            pltpu.SemaphoreType.DMA,
        ],
    )
    def emb(xtl_hbm, table_hbm, out_hbm, ibuf0, ibuf1, g0, g1, gp0, gp1,
            o0, o1, isem0, isem1, gsem0, gsem1, osem0, osem1):
        ibuf = (ibuf0, ibuf1)
        gb = (g0, g1)
        gp = (gp0, gp1)
        ob = (o0, o1)
        isem = (isem0, isem1)
        gsem = (gsem0, gsem1)
        osem = (osem0, osem1)

        wid = lax.axis_index("s") * NC + lax.axis_index("c")
        u0 = wid * U

        rowv = [
            lax.broadcasted_iota(jnp.int32, (LANES,), 0) + g * LANES
            for g in range(BB // LANES)
        ]

        def unit_su(u):
            ug = u0 + u
            return ug // NB, lax.rem(ug, NB)

        def issue_idx(u, b):
            s, bhi = unit_su(u)
            pltpu.async_copy(
                xtl_hbm.at[pl.ds(s * (NB * BB) + bhi * BB, BB)],
                ibuf[b], isem[b],
            )

        def wait_idx(u, b):
            s, bhi = unit_su(u)
            pltpu.make_async_copy(
                xtl_hbm.at[pl.ds(s * (NB * BB) + bhi * BB, BB)],
                ibuf[b], isem[b],
            ).wait()

        def issue_gather(b):
            pltpu.async_copy(table_hbm.at[ibuf[b]], gb[b], gsem[b])

        def wait_gather(b):
            pltpu.make_async_copy(
                table_hbm.at[ibuf[b]], gb[b], gsem[b]
            ).wait()

        def issue_out(u, b):
            s, bhi = unit_su(u)
            pltpu.async_copy(ob[b], out_hbm.at[s, :, bhi], osem[b])

        def wait_out(u, b):
            s, bhi = unit_su(u)
            pltpu.make_async_copy(
                ob[b], out_hbm.at[s, :, bhi], osem[b]
            ).wait()

        def transpose(b):
            @plsc.parallel_loop(0, BB, unroll=8)
            def rp(r):
                for j in range(D_MODEL // LANES):
                    sl = pl.ds(j * LANES, LANES)
                    gp[b][r, sl] = gb[b][r, sl]

            @plsc.parallel_loop(0, D_MODEL, unroll=8)
            def tp(d):
                dhi = lax.shift_right_logical(d, 3)
                dlo = lax.bitwise_and(d, 7)
                cv = jnp.full((LANES,), d, jnp.int32)
                for g in range(BB // LANES):
                    v = plsc.load_gather(gp[b], [rowv[g], cv])
                    ob[b][dhi, dlo, pl.ds(g * LANES, LANES)] = v * SCALE

        issue_idx(0, 0)
        issue_idx(1, 1)
        wait_idx(0, 0)
        issue_gather(0)
        for u in range(2):
            b = u % 2
            wait_gather(b)
            wait_idx(u + 1, 1 - b)
            issue_gather(1 - b)
            issue_idx(u + 2, b)
            transpose(b)
            issue_out(u, b)

        def pair(i, carry):
            for b in range(2):
                u = 2 * i + b
                wait_gather(b)
                wait_idx(u + 1, 1 - b)
                issue_gather(1 - b)
                issue_idx(u + 2, b)
                wait_out(u - 2, b)
                transpose(b)
                issue_out(u, b)
            return carry

        lax.fori_loop(1, U // 2 - 1, pair, 0)

        u = U - 2
        wait_gather(0)
        wait_idx(u + 1, 1)
        issue_gather(1)
        wait_out(u - 2, 0)
        transpose(0)
        issue_out(u, 0)

        u = U - 1
        wait_gather(1)
        wait_out(u - 2, 1)
        transpose(1)
        issue_out(u, 1)

        wait_out(U - 2, 0)
        wait_out(U - 1, 1)

    return emb


@functools.lru_cache(maxsize=None)
def _build_fmt(V):
    info = plsc.get_sparse_core_info()
    NC, NS = info.num_cores, info.num_subcores
    NW = NC * NS
    CH = 256
    NF = V // CH
    NU = NF + 1
    TAIL = V - NF * CH
    UPW = (NU + NW - 1) // NW
    assert TAIL == D_MODEL

    mesh = plsc.VectorSubcoreMesh(core_axis_name="c", subcore_axis_name="s")

    @functools.partial(
        pl.kernel,
        mesh=mesh,
        out_type=jax.ShapeDtypeStruct((V // 2, 2 * D_MODEL), jnp.float32),
        compiler_params=pltpu.CompilerParams(
            use_tc_tiling_on_sc=True, needs_layout_passes=False
        ),
        name="fmt",
        scratch_types=[
            pltpu.VMEM((CH, D_MODEL), jnp.float32),
            pltpu.VMEM((CH, D_MODEL), jnp.float32),
            pltpu.VMEM((CH // 2, 2 * D_MODEL), jnp.float32),
            pltpu.VMEM((CH // 2, 2 * D_MODEL), jnp.float32),
            pltpu.SemaphoreType.DMA,
            pltpu.SemaphoreType.DMA,
            pltpu.SemaphoreType.DMA,
            pltpu.SemaphoreType.DMA,
        ],
    )
    def fmt(tbl_hbm, scr_hbm, s0, s1, o0, o1, is0, is1, os0, os1):
        sb = (s0, s1)
        ob = (o0, o1)
        isem = (is0, is1)
        osem = (os0, os1)

        wid = lax.axis_index("s") * NC + lax.axis_index("c")

        def unit(i):
            return wid + NW * i

        def dma_in(u, b, issue):
            @pl.when(u < NF)
            def _():
                c = pltpu.async_copy if issue else (
                    lambda s, d, m: pltpu.make_async_copy(s, d, m).wait())
                c(tbl_hbm.at[pl.ds(u * CH, CH)], sb[b], isem[b])

            @pl.when(u == NF)
            def _():
                c = pltpu.async_copy if issue else (
                    lambda s, d, m: pltpu.make_async_copy(s, d, m).wait())
                c(
                    tbl_hbm.at[pl.ds(NF * CH, TAIL)],
                    sb[b].at[pl.ds(0, TAIL)],
                    isem[b],
                )

        def dma_out(u, b, issue):
            @pl.when(u < NF)
            def _():
                c = pltpu.async_copy if issue else (
                    lambda s, d, m: pltpu.make_async_copy(s, d, m).wait())
                c(ob[b], scr_hbm.at[pl.ds(u * (CH // 2), CH // 2)], osem[b])

            @pl.when(u == NF)
            def _():
                c = pltpu.async_copy if issue else (
                    lambda s, d, m: pltpu.make_async_copy(s, d, m).wait())
                c(
                    ob[b].at[pl.ds(0, TAIL // 2)],
                    scr_hbm.at[pl.ds(NF * (CH // 2), TAIL // 2)],
                    osem[b],
                )

        def compute(b):
            @plsc.parallel_loop(0, CH // 2, unroll=8)
            def rp(k):
                for j in range(D_MODEL // LANES):
                    sl = pl.ds(j * LANES, LANES)
                    ob[b][k, sl] = sb[b][2 * k, sl]
                    ob[b][k, pl.ds(D_MODEL + j * LANES, LANES)] = (
                        sb[b][2 * k + 1, sl]
                    )

        dma_in(unit(0), 0, True)
        dma_in(unit(1), 1, True)

        def pair(t, carry):
            for b in range(2):
                i = 2 * t + b
                u = unit(i)

                @pl.when(u < NU)
                def _():
                    dma_in(u, b, False)
                    compute(b)

                @pl.when(jnp.logical_and(i >= 2, unit(i - 2) < NU))
                def _():
                    dma_out(unit(i - 2), b, False)

                @pl.when(u < NU)
                def _():
                    dma_out(u, b, True)

                @pl.when(unit(i + 2) < NU)
                def _():
                    dma_in(unit(i + 2), b, True)
            return carry

        lax.fori_loop(0, (UPW + 3) // 2 + 1, pair, 0)

    return fmt


def kernel(x, table):
    B0, S = x.shape
    NB = B0 // BB
    V = table.shape[0]
    scr = _build_fmt(V)(table)
    tab_lin = scr.reshape(V, D_MODEL)
    xtl = jnp.transpose(x).reshape(-1).astype(jnp.int32)
    out5 = _build(S, NB)(xtl, tab_lin)
    out = jnp.transpose(out5, (2, 4, 0, 1, 3)).reshape(B0, S, D_MODEL)
    return out


# speedup vs baseline: 1.3488x; 1.0140x over previous
import functools
import math

import jax
import jax.numpy as jnp
from jax import lax
from jax.experimental import pallas as pl
from jax.experimental.pallas import tpu as pltpu
from jax.experimental.pallas import tpu_sc as plsc

D_MODEL = 64
SCALE = math.sqrt(D_MODEL)
LANES = 16
BB = 128


@functools.lru_cache(maxsize=None)
def _build(S, NB):
    info = plsc.get_sparse_core_info()
    NC, NS = info.num_cores, info.num_subcores
    NW = NC * NS
    n_units = S * NB
    assert n_units % NW == 0
    U = n_units // NW
    assert U % 2 == 0 and U >= 6

    mesh = plsc.VectorSubcoreMesh(core_axis_name="c", subcore_axis_name="s")

    @functools.partial(
        pl.kernel,
        mesh=mesh,
        out_type=jax.ShapeDtypeStruct(
            (S, D_MODEL // 8, NB, 8, BB), jnp.float32
        ),
        compiler_params=pltpu.CompilerParams(
            use_tc_tiling_on_sc=False, needs_layout_passes=False
        ),
        scratch_types=[
            pltpu.VMEM((BB,), jnp.int32),
            pltpu.VMEM((BB,), jnp.int32),
            pltpu.VMEM((BB, D_MODEL), jnp.float32),
            pltpu.VMEM((BB, D_MODEL), jnp.float32),
            pltpu.VMEM((BB, D_MODEL + 1), jnp.float32),
            pltpu.VMEM((BB, D_MODEL + 1), jnp.float32),
            pltpu.VMEM((D_MODEL // 8, 8, BB), jnp.float32),
            pltpu.VMEM((D_MODEL // 8, 8, BB), jnp.float32),
            pltpu.SemaphoreType.DMA,
            pltpu.SemaphoreType.DMA,
            pltpu.SemaphoreType.DMA,
            pltpu.SemaphoreType.DMA,
            pltpu.SemaphoreType.DMA,
            pltpu.SemaphoreType.DMA,
        ],
    )
    def emb(xtl_hbm, table_hbm, out_hbm, ibuf0, ibuf1, g0, g1, gp0, gp1,
            o0, o1, isem0, isem1, gsem0, gsem1, osem0, osem1):
        ibuf = (ibuf0, ibuf1)
        gb = (g0, g1)
        gp = (gp0, gp1)
        ob = (o0, o1)
        isem = (isem0, isem1)
        gsem = (gsem0, gsem1)
        osem = (osem0, osem1)

        wid = lax.axis_index("s") * NC + lax.axis_index("c")
        u0 = wid * U

        rowv = [
            lax.broadcasted_iota(jnp.int32, (LANES,), 0) + g * LANES
            for g in range(BB // LANES)
        ]

        def unit_su(u):
            ug = u0 + u
            return ug // NB, lax.rem(ug, NB)

        def issue_idx(u, b):
            s, bhi = unit_su(u)
            pltpu.async_copy(
                xtl_hbm.at[pl.ds(s * (NB * BB) + bhi * BB, BB)],
                ibuf[b], isem[b],
            )

        def wait_idx(u, b):
            s, bhi = unit_su(u)
            pltpu.make_async_copy(
                xtl_hbm.at[pl.ds(s * (NB * BB) + bhi * BB, BB)],
                ibuf[b], isem[b],
            ).wait()

        def issue_gather(b):
            pltpu.async_copy(table_hbm.at[ibuf[b]], gb[b], gsem[b])

        def wait_gather(b):
            pltpu.make_async_copy(
                table_hbm.at[ibuf[b]], gb[b], gsem[b]
            ).wait()

        def issue_out(u, b):
            s, bhi = unit_su(u)
            pltpu.async_copy(ob[b], out_hbm.at[s, :, bhi], osem[b])

        def wait_out(u, b):
            s, bhi = unit_su(u)
            pltpu.make_async_copy(
                ob[b], out_hbm.at[s, :, bhi], osem[b]
            ).wait()

        def transpose(b):
            @plsc.parallel_loop(0, BB, unroll=16)
            def rp(r):
                for j in range(D_MODEL // LANES):
                    sl = pl.ds(j * LANES, LANES)
                    gp[b][r, sl] = gb[b][r, sl]

            @plsc.parallel_loop(0, D_MODEL, unroll=16)
            def tp(d):
                dhi = lax.shift_right_logical(d, 3)
                dlo = lax.bitwise_and(d, 7)
                cv = jnp.full((LANES,), d, jnp.int32)
                for g in range(BB // LANES):
                    v = plsc.load_gather(gp[b], [rowv[g], cv])
                    ob[b][dhi, dlo, pl.ds(g * LANES, LANES)] = v * SCALE

        issue_idx(0, 0)
        issue_idx(1, 1)
        wait_idx(0, 0)
        issue_gather(0)
        for u in range(2):
            b = u % 2
            wait_gather(b)
            wait_idx(u + 1, 1 - b)
            issue_gather(1 - b)
            issue_idx(u + 2, b)
            transpose(b)
            issue_out(u, b)

        def pair(i, carry):
            for b in range(2):
                u = 2 * i + b
                wait_gather(b)
                wait_idx(u + 1, 1 - b)
                issue_gather(1 - b)
                issue_idx(u + 2, b)
                wait_out(u - 2, b)
                transpose(b)
                issue_out(u, b)
            return carry

        lax.fori_loop(1, U // 2 - 1, pair, 0)

        u = U - 2
        wait_gather(0)
        wait_idx(u + 1, 1)
        issue_gather(1)
        wait_out(u - 2, 0)
        transpose(0)
        issue_out(u, 0)

        u = U - 1
        wait_gather(1)
        wait_out(u - 2, 1)
        transpose(1)
        issue_out(u, 1)

        wait_out(U - 2, 0)
        wait_out(U - 1, 1)

    return emb


def kernel(x, table):
    B0, S = x.shape
    NB = B0 // BB
    xtl = jnp.transpose(x).reshape(-1).astype(jnp.int32)
    out5 = _build(S, NB)(xtl, table)
    out = jnp.transpose(out5, (2, 4, 0, 1, 3)).reshape(B0, S, D_MODEL)
    return out
